# Initial kernel scaffold; baseline (speedup 1.0000x reference)
import jax
import jax.numpy as jnp
from jax.experimental import pallas as pl


def kernel(x_s, x_t, edge_index, edge_attr, u, batch_s, W1a, b1a, W1b, b1b, W2a, b2a, W2b, b2b):
    raise NotImplementedError("write your pallas kernel here")


# speedup vs baseline: 5.8609x; 5.8609x over previous
import functools

import jax
import jax.numpy as jnp
from jax import lax
from jax.experimental import pallas as pl
from jax.experimental.pallas import tpu as pltpu
from jax.experimental.pallas import tpu_sc as plsc

F_XT = 5
F_E = 10
D1 = 15
LANES = 16
CHUNK = 1024
SUB = 128
N_BLK = 2000


def _leaky(x):
    return jnp.where(x >= 0, x, 0.1 * x)


def _prep_y_body(xt_ref, w1a_ref, y_ref):
    res = jnp.dot(xt_ref[...], w1a_ref[...][:F_XT, :],
                  preferred_element_type=jnp.float32)
    y_ref[...] = jnp.concatenate([res, jnp.zeros_like(res[:, :1])], axis=1)


def _msg_body(attr_ref, yg_ref, w1a_ref, b1a_ref, w1b_ref, b1b_ref, msg_ref):
    z = jnp.dot(attr_ref[...], w1a_ref[...][F_XT:, :],
                preferred_element_type=jnp.float32)
    h = _leaky(yg_ref[...][:, :D1] + z + b1a_ref[...])
    m = jnp.dot(h, w1b_ref[...], preferred_element_type=jnp.float32) + b1b_ref[...]
    msg_ref[...] = jnp.concatenate([m, jnp.zeros_like(m[:, :1])], axis=1)


def _final_body(xs_ref, p1_ref, p2_ref, p3_ref, p4_ref, bs_ref, u_ref,
                w2a_ref, b2a_ref, w2b_ref, b2b_ref, out_ref):
    p1 = p1_ref[0]
    cnt = p1[:, D1:D1 + 1]
    s1 = p1[:, :D1]
    s2 = p2_ref[0][:, :D1]
    s3 = p3_ref[0][:, :D1]
    s4 = p4_ref[0][:, :D1]
    denom = jnp.maximum(cnt, 1.0)
    mean = s1 / denom
    var = jnp.maximum(s2 / denom - mean * mean, 0.0)
    std = jnp.sqrt(var + 1e-6)
    m2 = mean * mean
    c3 = s3 - 3.0 * mean * s2 + 2.0 * m2 * mean * cnt
    c4 = s4 - 4.0 * mean * s3 + 6.0 * m2 * s2 - 3.0 * m2 * m2 * cnt
    std2 = std * std
    skew = (c3 / denom) / (std2 * std)
    kurt = (c4 / denom) / (std2 * std2)
    onehot = (bs_ref[...] == lax.broadcasted_iota(jnp.int32, (1, 16), 1))
    ub = jnp.dot(onehot.astype(jnp.float32), u_ref[...],
                 preferred_element_type=jnp.float32)
    h = jnp.concatenate([xs_ref[...], cnt, mean, std, skew, kurt, ub], axis=1)
    h1 = _leaky(jnp.dot(h, w2a_ref[...], preferred_element_type=jnp.float32)
                + b2a_ref[...])
    out_ref[...] = (jnp.dot(h1, w2b_ref[...], preferred_element_type=jnp.float32)
                    + b2b_ref[...])


def _sc_mesh():
    return plsc.VectorSubcoreMesh(core_axis_name="c", subcore_axis_name="s",
                                  num_cores=2, num_subcores=16)


def _gather_rows(y, tgt2d, e_pad):
    per_w = e_pad // 32
    n_chunks = per_w // CHUNK
    k_sub = CHUNK // SUB

    @functools.partial(
        pl.kernel,
        out_type=jax.ShapeDtypeStruct((e_pad, LANES), jnp.float32),
        mesh=_sc_mesh(),
        compiler_params=pltpu.CompilerParams(use_tc_tiling_on_sc=False),
        scratch_types=[
            pltpu.VMEM((k_sub, SUB), jnp.int32),
            pltpu.VMEM((CHUNK, LANES), jnp.float32),
            pltpu.SemaphoreType.DMA,
        ],
    )
    def k(y_hbm, tgt_hbm, out_hbm, idx_v, rows_v, sem):
        c = lax.axis_index("c")
        s = lax.axis_index("s")
        wid = s * 2 + c
        base = wid * per_w
        rbase = wid * (per_w // SUB)

        def chunk_body(i, carry):
            cb = base + i * CHUNK
            rb = rbase + i * k_sub
            pltpu.sync_copy(tgt_hbm.at[pl.ds(rb, k_sub), :], idx_v)
            descs = [
                pltpu.async_copy(y_hbm.at[idx_v.at[j]],
                                 rows_v.at[pl.ds(j * SUB, SUB), :], sem)
                for j in range(k_sub)
            ]
            for d in descs:
                d.wait()
            pltpu.sync_copy(rows_v, out_hbm.at[pl.ds(cb, CHUNK), :])
            return carry

        lax.fori_loop(0, n_chunks, chunk_body, 0)

    return k(y, tgt2d)


def _scatter_moments(msg, src2d, zeros, e_pad, n_acc, second_pass):
    per_t = e_pad // 16
    n_chunks = per_t // CHUNK
    k_sub = CHUNK // SUB
    stripe = n_acc // 16

    @functools.partial(
        pl.kernel,
        out_type=jax.ShapeDtypeStruct((2, n_acc, LANES), jnp.float32),
        mesh=_sc_mesh(),
        compiler_params=pltpu.CompilerParams(use_tc_tiling_on_sc=False),
        scratch_types=[
            pltpu.VMEM((k_sub, SUB), jnp.int32),
            pltpu.VMEM((CHUNK, LANES), jnp.float32),
            pltpu.VMEM_SHARED((n_acc, LANES), jnp.float32),
            pltpu.SemaphoreType.DMA,
        ],
    )
    def k(msg_hbm, src_hbm, zeros_hbm, out_hbm, idx_v, rows_v, acc, sem):
        c = lax.axis_index("c")
        s = lax.axis_index("s")
        is_c0 = c == 0
        lane15 = lax.iota(jnp.int32, LANES) == (LANES - 1)
        pltpu.sync_copy(zeros_hbm, acc.at[pl.ds(s * stripe, stripe), :])
        plsc.subcore_barrier()

        def chunk_body(i, carry):
            cb = s * per_t + i * CHUNK
            rb = s * (per_t // SUB) + i * k_sub
            pltpu.sync_copy(src_hbm.at[pl.ds(rb, k_sub), :], idx_v)
            pltpu.sync_copy(msg_hbm.at[pl.ds(cb, CHUNK), :], rows_v)

            def row_body(r, rcarry):
                v = rows_v[r]
                v2 = v * v
                if second_pass:
                    wa = v2 * v
                    wb = v2 * v2
                else:
                    wa = jnp.where(lane15, 1.0, v)
                    wb = v2
                rows_v[r] = jnp.where(is_c0, wa, wb)
                return rcarry

            lax.fori_loop(0, CHUNK, row_body, 0, unroll=8)
            descs = [
                pltpu.async_copy(rows_v.at[pl.ds(j * SUB, SUB), :],
                                 acc.at[idx_v.at[j]], sem, add=True)
                for j in range(k_sub)
            ]
            for d in descs:
                d.wait()
            return carry

        lax.fori_loop(0, n_chunks, chunk_body, 0)
        plsc.subcore_barrier()
        pltpu.sync_copy(acc.at[pl.ds(s * stripe, stripe), :],
                        out_hbm.at[c, pl.ds(s * stripe, stripe), :])

    return k(msg, src2d, zeros)


def kernel(x_s, x_t, edge_index, edge_attr, u, batch_s,
           W1a, b1a, W1b, b1b, W2a, b2a, W2b, b2b):
    n = x_s.shape[0]
    e = edge_attr.shape[0]
    e_pad = -(-e // (32 * CHUNK)) * (32 * CHUNK)
    n_acc = -(-(n + 256) // SUB) * SUB
    pad = e_pad - e

    src = edge_index[0]
    tgt = edge_index[1]
    if pad:
        trash = n + (jnp.arange(pad, dtype=jnp.int32) % (n_acc - n))
        src = jnp.concatenate([src, trash])
        tgt = jnp.concatenate([tgt, jnp.zeros((pad,), jnp.int32)])
        edge_attr = jnp.concatenate(
            [edge_attr, jnp.zeros((pad, F_E), jnp.float32)])
    src2d = src.reshape(e_pad // SUB, SUB)
    tgt2d = tgt.reshape(e_pad // SUB, SUB)

    n_grid = n // N_BLK
    full = lambda shape: pl.BlockSpec(shape, lambda i: tuple(0 for _ in shape))
    b1a2 = b1a.reshape(1, D1)
    b1b2 = b1b.reshape(1, D1)
    b2a2 = b2a.reshape(1, -1)
    b2b2 = b2b.reshape(1, -1)

    y = pl.pallas_call(
        _prep_y_body,
        grid=(n_grid,),
        in_specs=[pl.BlockSpec((N_BLK, F_XT), lambda i: (i, 0)),
                  full((D1, D1))],
        out_specs=pl.BlockSpec((N_BLK, LANES), lambda i: (i, 0)),
        out_shape=jax.ShapeDtypeStruct((n, LANES), jnp.float32),
    )(x_t, W1a)

    yg = _gather_rows(y, tgt2d, e_pad)

    e_blk = 2048
    e_grid = e_pad // e_blk
    msg = pl.pallas_call(
        _msg_body,
        grid=(e_grid,),
        in_specs=[pl.BlockSpec((e_blk, F_E), lambda i: (i, 0)),
                  pl.BlockSpec((e_blk, LANES), lambda i: (i, 0)),
                  full((D1, D1)), full((1, D1)), full((D1, D1)), full((1, D1))],
        out_specs=pl.BlockSpec((e_blk, LANES), lambda i: (i, 0)),
        out_shape=jax.ShapeDtypeStruct((e_pad, LANES), jnp.float32),
    )(edge_attr, yg, W1a, b1a2, W1b, b1b2)

    zeros = jnp.zeros((n_acc // 16, LANES), jnp.float32)
    p12 = _scatter_moments(msg, src2d, zeros, e_pad, n_acc, second_pass=False)
    p34 = _scatter_moments(msg, src2d, zeros, e_pad, n_acc, second_pass=True)

    bs2 = batch_s.reshape(n, 1)
    mom_spec = [pl.BlockSpec((1, N_BLK, LANES), lambda i, _j=j: (_j, i, 0))
                for j in (0, 1)]
    out = pl.pallas_call(
        _final_body,
        grid=(n_grid,),
        in_specs=[pl.BlockSpec((N_BLK, x_s.shape[1]), lambda i: (i, 0)),
                  mom_spec[0], mom_spec[1], mom_spec[0], mom_spec[1],
                  pl.BlockSpec((N_BLK, 1), lambda i: (i, 0)),
                  full(u.shape), full(W2a.shape), full((1, b2a.shape[0])),
                  full(W2b.shape), full((1, b2b.shape[0]))],
        out_specs=pl.BlockSpec((N_BLK, W2b.shape[1]), lambda i: (i, 0)),
        out_shape=jax.ShapeDtypeStruct((n, W2b.shape[1]), jnp.float32),
    )(x_s, p12, p12, p34, p34, bs2, u, W2a, b2a2, W2b, b2b2)
    return out


# speedup vs baseline: 11.9667x; 2.0418x over previous
import functools

import jax
import jax.numpy as jnp
from jax import lax
from jax.experimental import pallas as pl
from jax.experimental.pallas import tpu as pltpu
from jax.experimental.pallas import tpu_sc as plsc

F_XT = 5
F_E = 10
D1 = 15
LANES = 16
CHUNK = 1024
SUB = 128
N_BLK = 2000


def _leaky(x):
    return jnp.where(x >= 0, x, 0.1 * x)


def _kron8(w):
    return jnp.kron(jnp.eye(8, dtype=jnp.float32), w)


def _prep_y_body(xpk_ref, wy_ref, ypk_ref):
    ypk_ref[...] = jnp.dot(xpk_ref[...], wy_ref[...],
                           preferred_element_type=jnp.float32)


def _msg_body(attr_ref, ypk_ref, wz_ref, b1a_ref, wm_ref, b1b_ref, out_ref):
    z = jnp.dot(attr_ref[...], wz_ref[...], preferred_element_type=jnp.float32)
    h = _leaky(ypk_ref[...] + z + b1a_ref[...])
    out_ref[...] = (jnp.dot(h, wm_ref[...], preferred_element_type=jnp.float32)
                    + b1b_ref[...])


def _final_body(xs_ref, p1_ref, p2_ref, p3_ref, p4_ref, bs_ref, u_ref,
                w2a_ref, b2a_ref, w2b_ref, b2b_ref, out_ref):
    p1 = p1_ref[0]
    cnt = p1[:, D1:D1 + 1]
    s1 = p1[:, :D1]
    s2 = p2_ref[0][:, :D1]
    s3 = p3_ref[0][:, :D1]
    s4 = p4_ref[0][:, :D1]
    denom = jnp.maximum(cnt, 1.0)
    mean = s1 / denom
    var = jnp.maximum(s2 / denom - mean * mean, 0.0)
    std = jnp.sqrt(var + 1e-6)
    m2 = mean * mean
    c3 = s3 - 3.0 * mean * s2 + 2.0 * m2 * mean * cnt
    c4 = s4 - 4.0 * mean * s3 + 6.0 * m2 * s2 - 3.0 * m2 * m2 * cnt
    std2 = std * std
    skew = (c3 / denom) / (std2 * std)
    kurt = (c4 / denom) / (std2 * std2)
    onehot = (bs_ref[...] == lax.broadcasted_iota(jnp.int32, (1, 16), 1))
    ub = jnp.dot(onehot.astype(jnp.float32), u_ref[...],
                 preferred_element_type=jnp.float32)
    h = jnp.concatenate([xs_ref[...], cnt, mean, std, skew, kurt, ub], axis=1)
    h1 = _leaky(jnp.dot(h, w2a_ref[...], preferred_element_type=jnp.float32)
                + b2a_ref[...])
    out_ref[...] = (jnp.dot(h1, w2b_ref[...], preferred_element_type=jnp.float32)
                    + b2b_ref[...])


def _sc_mesh():
    return plsc.VectorSubcoreMesh(core_axis_name="c", subcore_axis_name="s",
                                  num_cores=2, num_subcores=16)


def _gather_rows(y, tgt2d, e_pad):
    per_w = e_pad // 32
    n_chunks = per_w // CHUNK
    k_sub = CHUNK // SUB

    @functools.partial(
        pl.kernel,
        out_type=jax.ShapeDtypeStruct((e_pad, LANES), jnp.float32),
        mesh=_sc_mesh(),
        compiler_params=pltpu.CompilerParams(use_tc_tiling_on_sc=False),
        scratch_types=[
            pltpu.VMEM((k_sub, SUB), jnp.int32),
            pltpu.VMEM((CHUNK, LANES), jnp.float32),
            pltpu.SemaphoreType.DMA,
        ],
    )
    def k(y_hbm, tgt_hbm, out_hbm, idx_v, rows_v, sem):
        c = lax.axis_index("c")
        s = lax.axis_index("s")
        wid = s * 2 + c
        base = wid * per_w
        rbase = wid * (per_w // SUB)

        def chunk_body(i, carry):
            cb = base + i * CHUNK
            rb = rbase + i * k_sub
            pltpu.sync_copy(tgt_hbm.at[pl.ds(rb, k_sub), :], idx_v)
            descs = [
                pltpu.async_copy(y_hbm.at[idx_v.at[j]],
                                 rows_v.at[pl.ds(j * SUB, SUB), :], sem)
                for j in range(k_sub)
            ]
            for d in descs:
                d.wait()
            pltpu.sync_copy(rows_v, out_hbm.at[pl.ds(cb, CHUNK), :])
            return carry

        lax.fori_loop(0, n_chunks, chunk_body, 0)

    return k(y, tgt2d)


def _scatter_moments(msg, src2d, zeros, e_pad, n_acc):
    per_t = e_pad // 16
    n_chunks = per_t // CHUNK
    k_sub = CHUNK // SUB
    stripe = n_acc // 16

    @functools.partial(
        pl.kernel,
        out_type=jax.ShapeDtypeStruct((4, n_acc, LANES), jnp.float32),
        mesh=_sc_mesh(),
        compiler_params=pltpu.CompilerParams(use_tc_tiling_on_sc=False),
        scratch_types=[
            pltpu.VMEM((k_sub, SUB), jnp.int32),
            pltpu.VMEM((CHUNK, LANES), jnp.float32),
            pltpu.VMEM_SHARED((n_acc, LANES), jnp.float32),
            pltpu.SemaphoreType.DMA,
        ],
    )
    def k(msg_hbm, src_hbm, zeros_hbm, out_hbm, idx_v, rows_v, acc, sem):
        c = lax.axis_index("c")
        s = lax.axis_index("s")
        is_c0 = c == 0
        lane15 = lax.iota(jnp.int32, LANES) == (LANES - 1)

        for sweep in range(2):
            pltpu.sync_copy(zeros_hbm, acc.at[pl.ds(s * stripe, stripe), :])
            plsc.subcore_barrier()

            def chunk_body(i, carry):
                cb = s * per_t + i * CHUNK
                rb = s * (per_t // SUB) + i * k_sub
                pltpu.sync_copy(src_hbm.at[pl.ds(rb, k_sub), :], idx_v)
                pltpu.sync_copy(msg_hbm.at[pl.ds(cb, CHUNK), :], rows_v)

                def row_body(r, rcarry):
                    v = rows_v[r]
                    v2 = v * v
                    if sweep:
                        wa = v2 * v
                        wb = v2 * v2
                    else:
                        wa = jnp.where(lane15, 1.0, v)
                        wb = v2
                    rows_v[r] = jnp.where(is_c0, wa, wb)
                    return rcarry

                lax.fori_loop(0, CHUNK, row_body, 0, unroll=8)
                descs = [
                    pltpu.async_copy(rows_v.at[pl.ds(j * SUB, SUB), :],
                                     acc.at[idx_v.at[j]], sem, add=True)
                    for j in range(k_sub)
                ]
                for d in descs:
                    d.wait()
                return carry

            lax.fori_loop(0, n_chunks, chunk_body, 0)
            plsc.subcore_barrier()
            pltpu.sync_copy(acc.at[pl.ds(s * stripe, stripe), :],
                            out_hbm.at[2 * sweep + c,
                                       pl.ds(s * stripe, stripe), :])

    return k(msg, src2d, zeros)


def kernel(x_s, x_t, edge_index, edge_attr, u, batch_s,
           W1a, b1a, W1b, b1b, W2a, b2a, W2b, b2b):
    n = x_s.shape[0]
    e = edge_attr.shape[0]
    e_pad = -(-e // (32 * CHUNK)) * (32 * CHUNK)
    epk = e_pad // 8
    n_acc = -(-(n + 256) // SUB) * SUB
    pad = e_pad - e

    src = edge_index[0]
    tgt = edge_index[1]
    if pad:
        trash = n + (jnp.arange(pad, dtype=jnp.int32) % (n_acc - n))
        src = jnp.concatenate([src, trash])
        tgt = jnp.concatenate([tgt, jnp.zeros((pad,), jnp.int32)])
    src2d = src.reshape(e_pad // SUB, SUB)
    tgt2d = tgt.reshape(e_pad // SUB, SUB)

    pad_c = lambda w: jnp.pad(w, ((0, 0), (0, LANES - w.shape[1])))
    wy = _kron8(pad_c(W1a[:F_XT]))
    wz = _kron8(pad_c(W1a[F_XT:]))
    wm = _kron8(jnp.pad(W1b, ((0, 1), (0, 1))))
    b1a_pk = jnp.tile(jnp.pad(b1a, (0, 1)), 8).reshape(1, 128)
    b1b_pk = jnp.tile(jnp.pad(b1b, (0, 1)), 8).reshape(1, 128)

    n_grid = n // N_BLK
    full = lambda shape: pl.BlockSpec(shape, lambda i: tuple(0 for _ in shape))
    b2a2 = b2a.reshape(1, -1)
    b2b2 = b2b.reshape(1, -1)

    n8 = n // 8
    xpk = x_t.reshape(n8, 8 * F_XT)
    ypk = pl.pallas_call(
        _prep_y_body,
        grid=(1,),
        in_specs=[pl.BlockSpec((n8, 8 * F_XT), lambda i: (0, 0)),
                  full((8 * F_XT, 128))],
        out_specs=pl.BlockSpec((n8, 128), lambda i: (0, 0)),
        out_shape=jax.ShapeDtypeStruct((n8, 128), jnp.float32),
    )(xpk, wy)
    y = ypk.reshape(n, LANES)

    yg = _gather_rows(y, tgt2d, e_pad)

    attr_pk = edge_attr.reshape(e // 8, 8 * F_E)
    ypk_e = yg.reshape(epk, 128)
    e_blk = 2048
    e_grid = epk // e_blk
    msg_pk = pl.pallas_call(
        _msg_body,
        grid=(e_grid,),
        in_specs=[pl.BlockSpec((e_blk, 8 * F_E), lambda i: (i, 0)),
                  pl.BlockSpec((e_blk, 128), lambda i: (i, 0)),
                  full((8 * F_E, 128)), full((1, 128)),
                  full((128, 128)), full((1, 128))],
        out_specs=pl.BlockSpec((e_blk, 128), lambda i: (i, 0)),
        out_shape=jax.ShapeDtypeStruct((epk, 128), jnp.float32),
    )(attr_pk, ypk_e, wz, b1a_pk, wm, b1b_pk)
    msg = msg_pk.reshape(e_pad, LANES)

    zeros = jnp.zeros((n_acc // 16, LANES), jnp.float32)
    p = _scatter_moments(msg, src2d, zeros, e_pad, n_acc)

    bs2 = batch_s.reshape(n, 1)
    mom_spec = [pl.BlockSpec((1, N_BLK, LANES), lambda i, _j=j: (_j, i, 0))
                for j in (0, 1, 2, 3)]
    out = pl.pallas_call(
        _final_body,
        grid=(n_grid,),
        in_specs=[pl.BlockSpec((N_BLK, x_s.shape[1]), lambda i: (i, 0)),
                  mom_spec[0], mom_spec[1], mom_spec[2], mom_spec[3],
                  pl.BlockSpec((N_BLK, 1), lambda i: (i, 0)),
                  full(u.shape), full(W2a.shape), full((1, b2a.shape[0])),
                  full(W2b.shape), full((1, b2b.shape[0]))],
        out_specs=pl.BlockSpec((N_BLK, W2b.shape[1]), lambda i: (i, 0)),
        out_shape=jax.ShapeDtypeStruct((n, W2b.shape[1]), jnp.float32),
    )(x_s, p, p, p, p, bs2, u, W2a, b2a2, W2b, b2b2)
    return out


# speedup vs baseline: 12.0921x; 1.0105x over previous
import functools

import jax
import jax.numpy as jnp
from jax import lax
from jax.experimental import pallas as pl
from jax.experimental.pallas import tpu as pltpu
from jax.experimental.pallas import tpu_sc as plsc

F_XT = 5
F_E = 10
D1 = 15
LANES = 16
CHUNK = 1024
SUB = 128
N_BLK = 2000


def _leaky(x):
    return jnp.where(x >= 0, x, 0.1 * x)


def _kron8(w):
    return jnp.kron(jnp.eye(8, dtype=jnp.float32), w)


def _prep_y_body(xpk_ref, wy_ref, ypk_ref):
    ypk_ref[...] = jnp.dot(xpk_ref[...], wy_ref[...],
                           preferred_element_type=jnp.float32)


def _msg_body(attr_ref, ypk_ref, wz_ref, b1a_ref, wm_ref, b1b_ref, out_ref):
    z = jnp.dot(attr_ref[...], wz_ref[...], preferred_element_type=jnp.float32)
    h = _leaky(ypk_ref[...] + z + b1a_ref[...])
    out_ref[...] = (jnp.dot(h, wm_ref[...], preferred_element_type=jnp.float32)
                    + b1b_ref[...])


def _final_body(xs_ref, p1_ref, p2_ref, p3_ref, p4_ref, bs_ref, u_ref,
                w2a_ref, b2a_ref, w2b_ref, b2b_ref, out_ref):
    p1 = p1_ref[0]
    cnt = p1[:, D1:D1 + 1]
    s1 = p1[:, :D1]
    s2 = p2_ref[0][:, :D1]
    s3 = p3_ref[0][:, :D1]
    s4 = p4_ref[0][:, :D1]
    denom = jnp.maximum(cnt, 1.0)
    mean = s1 / denom
    var = jnp.maximum(s2 / denom - mean * mean, 0.0)
    std = jnp.sqrt(var + 1e-6)
    m2 = mean * mean
    c3 = s3 - 3.0 * mean * s2 + 2.0 * m2 * mean * cnt
    c4 = s4 - 4.0 * mean * s3 + 6.0 * m2 * s2 - 3.0 * m2 * m2 * cnt
    std2 = std * std
    skew = (c3 / denom) / (std2 * std)
    kurt = (c4 / denom) / (std2 * std2)
    onehot = (bs_ref[...] == lax.broadcasted_iota(jnp.int32, (1, 16), 1))
    ub = jnp.dot(onehot.astype(jnp.float32), u_ref[...],
                 preferred_element_type=jnp.float32)
    h = jnp.concatenate([xs_ref[...], cnt, mean, std, skew, kurt, ub], axis=1)
    h1 = _leaky(jnp.dot(h, w2a_ref[...], preferred_element_type=jnp.float32)
                + b2a_ref[...])
    out_ref[...] = (jnp.dot(h1, w2b_ref[...], preferred_element_type=jnp.float32)
                    + b2b_ref[...])


def _sc_mesh():
    return plsc.VectorSubcoreMesh(core_axis_name="c", subcore_axis_name="s",
                                  num_cores=2, num_subcores=16)


def _gather_rows(y, tgt2d, e_pad):
    per_w = e_pad // 32
    n_chunks = per_w // CHUNK
    k_sub = CHUNK // SUB

    @functools.partial(
        pl.kernel,
        out_type=jax.ShapeDtypeStruct((e_pad, LANES), jnp.float32),
        mesh=_sc_mesh(),
        compiler_params=pltpu.CompilerParams(use_tc_tiling_on_sc=False),
        scratch_types=[
            pltpu.VMEM((k_sub, SUB), jnp.int32),
            pltpu.VMEM((CHUNK, LANES), jnp.float32),
            pltpu.SemaphoreType.DMA,
        ],
    )
    def k(y_hbm, tgt_hbm, out_hbm, idx_v, rows_v, sem):
        c = lax.axis_index("c")
        s = lax.axis_index("s")
        wid = s * 2 + c
        base = wid * per_w
        rbase = wid * (per_w // SUB)

        def chunk_body(i, carry):
            cb = base + i * CHUNK
            rb = rbase + i * k_sub
            pltpu.sync_copy(tgt_hbm.at[pl.ds(rb, k_sub), :], idx_v)
            descs = [
                pltpu.async_copy(y_hbm.at[idx_v.at[j]],
                                 rows_v.at[pl.ds(j * SUB, SUB), :], sem)
                for j in range(k_sub)
            ]
            for d in descs:
                d.wait()
            pltpu.sync_copy(rows_v, out_hbm.at[pl.ds(cb, CHUNK), :])
            return carry

        lax.fori_loop(0, n_chunks, chunk_body, 0)

    return k(y, tgt2d)


def _scatter_moments(msg, src2d, zeros, e_pad, n_acc):
    per_t = e_pad // 16
    n_chunks = per_t // CHUNK
    k_sub = CHUNK // SUB
    stripe = n_acc // 16

    @functools.partial(
        pl.kernel,
        out_type=jax.ShapeDtypeStruct((4, n_acc, LANES), jnp.float32),
        mesh=_sc_mesh(),
        compiler_params=pltpu.CompilerParams(use_tc_tiling_on_sc=False),
        scratch_types=[
            pltpu.VMEM((k_sub, SUB), jnp.int32),
            pltpu.VMEM((CHUNK, LANES), jnp.float32),
            pltpu.VMEM_SHARED((n_acc, LANES), jnp.float32),
            pltpu.SemaphoreType.DMA,
        ],
    )
    def k(msg_hbm, src_hbm, zeros_hbm, out_hbm, idx_v, rows_v, acc, sem):
        c = lax.axis_index("c")
        s = lax.axis_index("s")
        is_c0 = c == 0

        for sweep in range(2):
            pltpu.sync_copy(zeros_hbm, acc.at[pl.ds(s * stripe, stripe), :])
            plsc.subcore_barrier()

            def chunk_body(i, carry):
                cb = s * per_t + i * CHUNK
                rb = s * (per_t // SUB) + i * k_sub
                pltpu.sync_copy(src_hbm.at[pl.ds(rb, k_sub), :], idx_v)
                pltpu.sync_copy(msg_hbm.at[pl.ds(cb, CHUNK), :], rows_v)

                def power_loop(fn):
                    def row_body(r, rcarry):
                        rows_v[r] = fn(rows_v[r])
                        return rcarry
                    lax.fori_loop(0, CHUNK, row_body, 0, unroll=8)

                if sweep == 0:
                    @pl.when(jnp.logical_not(is_c0))
                    def _():
                        power_loop(lambda v: v * v)
                else:
                    @pl.when(is_c0)
                    def _():
                        power_loop(lambda v: v * v * v)

                    @pl.when(jnp.logical_not(is_c0))
                    def _():
                        def quad(v):
                            v2 = v * v
                            return v2 * v2
                        power_loop(quad)

                descs = [
                    pltpu.async_copy(rows_v.at[pl.ds(j * SUB, SUB), :],
                                     acc.at[idx_v.at[j]], sem, add=True)
                    for j in range(k_sub)
                ]
                for d in descs:
                    d.wait()
                return carry

            lax.fori_loop(0, n_chunks, chunk_body, 0)
            plsc.subcore_barrier()
            pltpu.sync_copy(acc.at[pl.ds(s * stripe, stripe), :],
                            out_hbm.at[2 * sweep + c,
                                       pl.ds(s * stripe, stripe), :])

    return k(msg, src2d, zeros)


def kernel(x_s, x_t, edge_index, edge_attr, u, batch_s,
           W1a, b1a, W1b, b1b, W2a, b2a, W2b, b2b):
    n = x_s.shape[0]
    e = edge_attr.shape[0]
    e_pad = -(-e // (32 * CHUNK)) * (32 * CHUNK)
    epk = e_pad // 8
    n_acc = -(-(n + 256) // SUB) * SUB
    pad = e_pad - e

    src = edge_index[0]
    tgt = edge_index[1]
    if pad:
        trash = n + (jnp.arange(pad, dtype=jnp.int32) % (n_acc - n))
        src = jnp.concatenate([src, trash])
        tgt = jnp.concatenate([tgt, jnp.zeros((pad,), jnp.int32)])
    src2d = src.reshape(e_pad // SUB, SUB)
    tgt2d = tgt.reshape(e_pad // SUB, SUB)

    pad_c = lambda w: jnp.pad(w, ((0, 0), (0, LANES - w.shape[1])))
    wy = _kron8(pad_c(W1a[:F_XT]))
    wz = _kron8(pad_c(W1a[F_XT:]))
    wm = _kron8(jnp.pad(W1b, ((0, 1), (0, 1))))
    b1a_pk = jnp.tile(jnp.pad(b1a, (0, 1)), 8).reshape(1, 128)
    b1b_pk = jnp.tile(jnp.pad(b1b, (0, 1), constant_values=1.0), 8).reshape(1, 128)

    n_grid = n // N_BLK
    full = lambda shape: pl.BlockSpec(shape, lambda i: tuple(0 for _ in shape))
    b2a2 = b2a.reshape(1, -1)
    b2b2 = b2b.reshape(1, -1)

    n8 = n // 8
    xpk = x_t.reshape(n8, 8 * F_XT)
    ypk = pl.pallas_call(
        _prep_y_body,
        grid=(1,),
        in_specs=[pl.BlockSpec((n8, 8 * F_XT), lambda i: (0, 0)),
                  full((8 * F_XT, 128))],
        out_specs=pl.BlockSpec((n8, 128), lambda i: (0, 0)),
        out_shape=jax.ShapeDtypeStruct((n8, 128), jnp.float32),
    )(xpk, wy)
    y = ypk.reshape(n, LANES)

    yg = _gather_rows(y, tgt2d, e_pad)

    attr_pk = edge_attr.reshape(e // 8, 8 * F_E)
    ypk_e = yg.reshape(epk, 128)
    e_blk = 2048
    e_grid = epk // e_blk
    msg_pk = pl.pallas_call(
        _msg_body,
        grid=(e_grid,),
        in_specs=[pl.BlockSpec((e_blk, 8 * F_E), lambda i: (i, 0)),
                  pl.BlockSpec((e_blk, 128), lambda i: (i, 0)),
                  full((8 * F_E, 128)), full((1, 128)),
                  full((128, 128)), full((1, 128))],
        out_specs=pl.BlockSpec((e_blk, 128), lambda i: (i, 0)),
        out_shape=jax.ShapeDtypeStruct((epk, 128), jnp.float32),
    )(attr_pk, ypk_e, wz, b1a_pk, wm, b1b_pk)
    msg = msg_pk.reshape(e_pad, LANES)

    zeros = jnp.zeros((n_acc // 16, LANES), jnp.float32)
    p = _scatter_moments(msg, src2d, zeros, e_pad, n_acc)

    bs2 = batch_s.reshape(n, 1)
    mom_spec = [pl.BlockSpec((1, N_BLK, LANES), lambda i, _j=j: (_j, i, 0))
                for j in (0, 1, 2, 3)]
    out = pl.pallas_call(
        _final_body,
        grid=(n_grid,),
        in_specs=[pl.BlockSpec((N_BLK, x_s.shape[1]), lambda i: (i, 0)),
                  mom_spec[0], mom_spec[1], mom_spec[2], mom_spec[3],
                  pl.BlockSpec((N_BLK, 1), lambda i: (i, 0)),
                  full(u.shape), full(W2a.shape), full((1, b2a.shape[0])),
                  full(W2b.shape), full((1, b2b.shape[0]))],
        out_specs=pl.BlockSpec((N_BLK, W2b.shape[1]), lambda i: (i, 0)),
        out_shape=jax.ShapeDtypeStruct((n, W2b.shape[1]), jnp.float32),
    )(x_s, p, p, p, p, bs2, u, W2a, b2a2, W2b, b2b2)
    return out


# speedup vs baseline: 13.6741x; 1.1308x over previous
import functools

import jax
import jax.numpy as jnp
from jax import lax
from jax.experimental import pallas as pl
from jax.experimental.pallas import tpu as pltpu
from jax.experimental.pallas import tpu_sc as plsc

F_XT = 5
F_E = 10
D1 = 15
LANES = 16
CHUNK = 1024
SUB = 128
N_BLK = 2000


def _leaky(x):
    return jnp.where(x >= 0, x, 0.1 * x)


def _kron8(w):
    return jnp.kron(jnp.eye(8, dtype=jnp.float32), w)


def _prep_y_body(xpk_ref, wy_ref, ypk_ref):
    ypk_ref[...] = jnp.dot(xpk_ref[...], wy_ref[...],
                           preferred_element_type=jnp.float32)


def _msg_body(attr_ref, ypk_ref, wz_ref, b1a_ref, wm_ref, b1b_ref, out_ref):
    z = jnp.dot(attr_ref[...], wz_ref[...], preferred_element_type=jnp.float32)
    h = _leaky(ypk_ref[...] + z + b1a_ref[...])
    out_ref[...] = (jnp.dot(h, wm_ref[...], preferred_element_type=jnp.float32)
                    + b1b_ref[...])


def _final_body(xs_ref, p1_ref, p2_ref, p3_ref, p4_ref, bs_ref, u_ref,
                w2a_ref, b2a_ref, w2b_ref, b2b_ref, out_ref):
    p1 = p1_ref[0]
    cnt = p1[:, D1:D1 + 1]
    s1 = p1[:, :D1]
    s2 = p2_ref[0][:, :D1]
    s3 = p3_ref[0][:, :D1]
    s4 = p4_ref[0][:, :D1]
    denom = jnp.maximum(cnt, 1.0)
    mean = s1 / denom
    var = jnp.maximum(s2 / denom - mean * mean, 0.0)
    std = jnp.sqrt(var + 1e-6)
    m2 = mean * mean
    c3 = s3 - 3.0 * mean * s2 + 2.0 * m2 * mean * cnt
    c4 = s4 - 4.0 * mean * s3 + 6.0 * m2 * s2 - 3.0 * m2 * m2 * cnt
    std2 = std * std
    skew = (c3 / denom) / (std2 * std)
    kurt = (c4 / denom) / (std2 * std2)
    onehot = (bs_ref[...] == lax.broadcasted_iota(jnp.int32, (1, 16), 1))
    ub = jnp.dot(onehot.astype(jnp.float32), u_ref[...],
                 preferred_element_type=jnp.float32)
    h = jnp.concatenate([xs_ref[...], cnt, mean, std, skew, kurt, ub], axis=1)
    h1 = _leaky(jnp.dot(h, w2a_ref[...], preferred_element_type=jnp.float32)
                + b2a_ref[...])
    out_ref[...] = (jnp.dot(h1, w2b_ref[...], preferred_element_type=jnp.float32)
                    + b2b_ref[...])


def _sc_mesh():
    return plsc.VectorSubcoreMesh(core_axis_name="c", subcore_axis_name="s",
                                  num_cores=2, num_subcores=16)


def _gather_rows(y, tgt2d, e_pad):
    per_w = e_pad // 32
    n_chunks = per_w // CHUNK
    k_sub = CHUNK // SUB

    @functools.partial(
        pl.kernel,
        out_type=jax.ShapeDtypeStruct((e_pad, LANES), jnp.float32),
        mesh=_sc_mesh(),
        compiler_params=pltpu.CompilerParams(use_tc_tiling_on_sc=False),
        scratch_types=[
            pltpu.VMEM((k_sub, SUB), jnp.int32),
            pltpu.VMEM((CHUNK, LANES), jnp.float32),
            pltpu.SemaphoreType.DMA,
        ],
    )
    def k(y_hbm, tgt_hbm, out_hbm, idx_v, rows_v, sem):
        c = lax.axis_index("c")
        s = lax.axis_index("s")
        wid = s * 2 + c
        base = wid * per_w
        rbase = wid * (per_w // SUB)

        def chunk_body(i, carry):
            cb = base + i * CHUNK
            rb = rbase + i * k_sub
            pltpu.sync_copy(tgt_hbm.at[pl.ds(rb, k_sub), :], idx_v)
            descs = [
                pltpu.async_copy(y_hbm.at[idx_v.at[j]],
                                 rows_v.at[pl.ds(j * SUB, SUB), :], sem)
                for j in range(k_sub)
            ]
            for d in descs:
                d.wait()
            pltpu.sync_copy(rows_v, out_hbm.at[pl.ds(cb, CHUNK), :])
            return carry

        lax.fori_loop(0, n_chunks, chunk_body, 0)

    return k(y, tgt2d)


def _scatter_moments(msg, src2d, zeros, e_pad, n_acc):
    chunk = CHUNK // 2
    per_t = e_pad // 16
    n_chunks = per_t // chunk
    k_sub = chunk // SUB
    stripe = n_acc // 16

    @functools.partial(
        pl.kernel,
        out_type=jax.ShapeDtypeStruct((4, n_acc, LANES), jnp.float32),
        mesh=_sc_mesh(),
        compiler_params=pltpu.CompilerParams(use_tc_tiling_on_sc=False),
        scratch_types=[
            pltpu.VMEM((2, k_sub, SUB), jnp.int32),
            pltpu.VMEM((2, chunk, LANES), jnp.float32),
            pltpu.VMEM_SHARED((n_acc, LANES), jnp.float32),
            pltpu.SemaphoreType.DMA,
            pltpu.SemaphoreType.DMA,
            pltpu.SemaphoreType.DMA,
        ],
    )
    def k(msg_hbm, src_hbm, zeros_hbm, out_hbm, idx_v, rows_v, acc, sem,
          lsem0, lsem1):
        c = lax.axis_index("c")
        s = lax.axis_index("s")
        is_c0 = c == 0
        lsem = (lsem0, lsem1)

        def load_pair(i, b, issue):
            cb = s * per_t + i * chunk
            rb = s * (per_t // SUB) + i * k_sub
            f = pltpu.async_copy if issue else (
                lambda a, d, sm: pltpu.make_async_copy(a, d, sm))
            return (f(src_hbm.at[pl.ds(rb, k_sub), :], idx_v.at[b], lsem[b]),
                    f(msg_hbm.at[pl.ds(cb, chunk), :], rows_v.at[b], lsem[b]))

        for sweep in range(2):
            pltpu.sync_copy(zeros_hbm, acc.at[pl.ds(s * stripe, stripe), :])
            plsc.subcore_barrier()
            load_pair(0, 0, True)

            def outer_body(i0, carry):
                for b in range(2):
                    i = 2 * i0 + b
                    for d in load_pair(i, b, False):
                        d.wait()

                    @pl.when(i + 1 < n_chunks)
                    def _():
                        load_pair(i + 1, 1 - b, True)

                    def power_loop(fn):
                        def row_body(r, rcarry):
                            rows_v[b, r] = fn(rows_v[b, r])
                            return rcarry
                        lax.fori_loop(0, chunk, row_body, 0, unroll=8)

                    if sweep == 0:
                        @pl.when(jnp.logical_not(is_c0))
                        def _():
                            power_loop(lambda v: v * v)
                    else:
                        @pl.when(is_c0)
                        def _():
                            power_loop(lambda v: v * v * v)

                        @pl.when(jnp.logical_not(is_c0))
                        def _():
                            def quad(v):
                                v2 = v * v
                                return v2 * v2
                            power_loop(quad)

                    descs = [
                        pltpu.async_copy(rows_v.at[b, pl.ds(j * SUB, SUB), :],
                                         acc.at[idx_v.at[b, j]], sem, add=True)
                        for j in range(k_sub)
                    ]
                    for d in descs:
                        d.wait()
                return carry

            lax.fori_loop(0, n_chunks // 2, outer_body, 0)
            plsc.subcore_barrier()
            pltpu.sync_copy(acc.at[pl.ds(s * stripe, stripe), :],
                            out_hbm.at[2 * sweep + c,
                                       pl.ds(s * stripe, stripe), :])

    return k(msg, src2d, zeros)


def kernel(x_s, x_t, edge_index, edge_attr, u, batch_s,
           W1a, b1a, W1b, b1b, W2a, b2a, W2b, b2b):
    n = x_s.shape[0]
    e = edge_attr.shape[0]
    e_pad = -(-e // (32 * CHUNK)) * (32 * CHUNK)
    epk = e_pad // 8
    n_acc = -(-(n + 256) // SUB) * SUB
    pad = e_pad - e

    src = edge_index[0]
    tgt = edge_index[1]
    if pad:
        trash = n + (jnp.arange(pad, dtype=jnp.int32) % (n_acc - n))
        src = jnp.concatenate([src, trash])
        tgt = jnp.concatenate([tgt, jnp.zeros((pad,), jnp.int32)])
    src2d = src.reshape(e_pad // SUB, SUB)
    tgt2d = tgt.reshape(e_pad // SUB, SUB)

    pad_c = lambda w: jnp.pad(w, ((0, 0), (0, LANES - w.shape[1])))
    wy = _kron8(pad_c(W1a[:F_XT]))
    wz = _kron8(pad_c(W1a[F_XT:]))
    wm = _kron8(jnp.pad(W1b, ((0, 1), (0, 1))))
    b1a_pk = jnp.tile(jnp.pad(b1a, (0, 1)), 8).reshape(1, 128)
    b1b_pk = jnp.tile(jnp.pad(b1b, (0, 1), constant_values=1.0), 8).reshape(1, 128)

    n_grid = n // N_BLK
    full = lambda shape: pl.BlockSpec(shape, lambda i: tuple(0 for _ in shape))
    b2a2 = b2a.reshape(1, -1)
    b2b2 = b2b.reshape(1, -1)

    n8 = n // 8
    xpk = x_t.reshape(n8, 8 * F_XT)
    ypk = pl.pallas_call(
        _prep_y_body,
        grid=(1,),
        in_specs=[pl.BlockSpec((n8, 8 * F_XT), lambda i: (0, 0)),
                  full((8 * F_XT, 128))],
        out_specs=pl.BlockSpec((n8, 128), lambda i: (0, 0)),
        out_shape=jax.ShapeDtypeStruct((n8, 128), jnp.float32),
    )(xpk, wy)
    y = ypk.reshape(n, LANES)

    yg = _gather_rows(y, tgt2d, e_pad)

    attr_pk = edge_attr.reshape(e // 8, 8 * F_E)
    ypk_e = yg.reshape(epk, 128)
    e_blk = 2048
    e_grid = epk // e_blk
    msg_pk = pl.pallas_call(
        _msg_body,
        grid=(e_grid,),
        in_specs=[pl.BlockSpec((e_blk, 8 * F_E), lambda i: (i, 0)),
                  pl.BlockSpec((e_blk, 128), lambda i: (i, 0)),
                  full((8 * F_E, 128)), full((1, 128)),
                  full((128, 128)), full((1, 128))],
        out_specs=pl.BlockSpec((e_blk, 128), lambda i: (i, 0)),
        out_shape=jax.ShapeDtypeStruct((epk, 128), jnp.float32),
    )(attr_pk, ypk_e, wz, b1a_pk, wm, b1b_pk)
    msg = msg_pk.reshape(e_pad, LANES)

    zeros = jnp.zeros((n_acc // 16, LANES), jnp.float32)
    p = _scatter_moments(msg, src2d, zeros, e_pad, n_acc)

    bs2 = batch_s.reshape(n, 1)
    mom_spec = [pl.BlockSpec((1, N_BLK, LANES), lambda i, _j=j: (_j, i, 0))
                for j in (0, 1, 2, 3)]
    out = pl.pallas_call(
        _final_body,
        grid=(n_grid,),
        in_specs=[pl.BlockSpec((N_BLK, x_s.shape[1]), lambda i: (i, 0)),
                  mom_spec[0], mom_spec[1], mom_spec[2], mom_spec[3],
                  pl.BlockSpec((N_BLK, 1), lambda i: (i, 0)),
                  full(u.shape), full(W2a.shape), full((1, b2a.shape[0])),
                  full(W2b.shape), full((1, b2b.shape[0]))],
        out_specs=pl.BlockSpec((N_BLK, W2b.shape[1]), lambda i: (i, 0)),
        out_shape=jax.ShapeDtypeStruct((n, W2b.shape[1]), jnp.float32),
    )(x_s, p, p, p, p, bs2, u, W2a, b2a2, W2b, b2b2)
    return out


# speedup vs baseline: 15.8222x; 1.1571x over previous
import functools

import jax
import jax.numpy as jnp
from jax import lax
from jax.experimental import pallas as pl
from jax.experimental.pallas import tpu as pltpu
from jax.experimental.pallas import tpu_sc as plsc

F_XT = 5
F_E = 10
D1 = 15
LANES = 16
CHUNK = 1024
SUB = 128
N_BLK = 2000


def _leaky(x):
    return jnp.where(x >= 0, x, 0.1 * x)


def _kron8(w):
    return jnp.kron(jnp.eye(8, dtype=jnp.float32), w)


def _prep_y_body(xpk_ref, wy_ref, ypk_ref):
    ypk_ref[...] = jnp.dot(xpk_ref[...], wy_ref[...],
                           preferred_element_type=jnp.float32)


def _msg_body(attr_ref, ypk_ref, wz_ref, b1a_ref, wm_ref, b1b_ref, out_ref):
    z = jnp.dot(attr_ref[...], wz_ref[...], preferred_element_type=jnp.float32)
    h = _leaky(ypk_ref[...] + z + b1a_ref[...])
    out_ref[...] = (jnp.dot(h, wm_ref[...], preferred_element_type=jnp.float32)
                    + b1b_ref[...])


def _final_body(xs_ref, p1_ref, p2_ref, p3_ref, p4_ref, bs_ref,
                selcnt_ref, kcnt_ref, kmean_ref, kstd_ref, kskew_ref,
                kkurt_ref, kxs_ref, krep_ref, iota_ref, kub_ref,
                b2a_ref, kw2b_ref, b2b_ref, out_ref):
    dot = functools.partial(jnp.dot, preferred_element_type=jnp.float32)
    p1 = p1_ref[0]
    p2 = p2_ref[0]
    p3 = p3_ref[0]
    p4 = p4_ref[0]
    cnt = dot(p1, selcnt_ref[...])
    denom = jnp.maximum(cnt, 1.0)
    mean = p1 / denom
    var = jnp.maximum(p2 / denom - mean * mean, 0.0)
    std = jnp.sqrt(var + 1e-6)
    m2 = mean * mean
    c3 = p3 - 3.0 * mean * p2 + 2.0 * m2 * mean * cnt
    c4 = p4 - 4.0 * mean * p3 + 6.0 * m2 * p2 - 3.0 * m2 * m2 * cnt
    std2 = std * std
    skew = (c3 / denom) / (std2 * std)
    kurt = (c4 / denom) / (std2 * std2)
    bs_rep = dot(bs_ref[...], krep_ref[...])
    onehot = (bs_rep == iota_ref[...]).astype(jnp.float32)
    pre = (dot(xs_ref[...], kxs_ref[...]) + dot(p1, kcnt_ref[...])
           + dot(mean, kmean_ref[...]) + dot(std, kstd_ref[...])
           + dot(skew, kskew_ref[...]) + dot(kurt, kkurt_ref[...])
           + dot(onehot, kub_ref[...]) + b2a_ref[...])
    h1 = _leaky(pre)
    out_ref[...] = dot(h1, kw2b_ref[...]) + b2b_ref[...]


def _sc_mesh():
    return plsc.VectorSubcoreMesh(core_axis_name="c", subcore_axis_name="s",
                                  num_cores=2, num_subcores=16)


def _gather_rows(y, tgt2d, e_pad):
    per_w = e_pad // 32
    n_chunks = per_w // CHUNK
    k_sub = CHUNK // SUB

    @functools.partial(
        pl.kernel,
        out_type=jax.ShapeDtypeStruct((e_pad, LANES), jnp.float32),
        mesh=_sc_mesh(),
        compiler_params=pltpu.CompilerParams(use_tc_tiling_on_sc=False),
        scratch_types=[
            pltpu.VMEM((k_sub, SUB), jnp.int32),
            pltpu.VMEM((CHUNK, LANES), jnp.float32),
            pltpu.SemaphoreType.DMA,
        ],
    )
    def k(y_hbm, tgt_hbm, out_hbm, idx_v, rows_v, sem):
        c = lax.axis_index("c")
        s = lax.axis_index("s")
        wid = s * 2 + c
        base = wid * per_w
        rbase = wid * (per_w // SUB)

        def chunk_body(i, carry):
            cb = base + i * CHUNK
            rb = rbase + i * k_sub
            pltpu.sync_copy(tgt_hbm.at[pl.ds(rb, k_sub), :], idx_v)
            descs = [
                pltpu.async_copy(y_hbm.at[idx_v.at[j]],
                                 rows_v.at[pl.ds(j * SUB, SUB), :], sem)
                for j in range(k_sub)
            ]
            for d in descs:
                d.wait()
            pltpu.sync_copy(rows_v, out_hbm.at[pl.ds(cb, CHUNK), :])
            return carry

        lax.fori_loop(0, n_chunks, chunk_body, 0)

    return k(y, tgt2d)


def _scatter_moments(msg, src2d, zeros, e_pad, n_acc):
    chunk = CHUNK // 2
    per_t = e_pad // 16
    n_chunks = per_t // chunk
    k_sub = chunk // SUB
    stripe = n_acc // 16

    @functools.partial(
        pl.kernel,
        out_type=jax.ShapeDtypeStruct((4, n_acc, LANES), jnp.float32),
        mesh=_sc_mesh(),
        compiler_params=pltpu.CompilerParams(use_tc_tiling_on_sc=False),
        scratch_types=[
            pltpu.VMEM((2, k_sub, SUB), jnp.int32),
            pltpu.VMEM((2, chunk, LANES), jnp.float32),
            pltpu.VMEM_SHARED((n_acc, LANES), jnp.float32),
            pltpu.SemaphoreType.DMA,
            pltpu.SemaphoreType.DMA,
            pltpu.SemaphoreType.DMA,
        ],
    )
    def k(msg_hbm, src_hbm, zeros_hbm, out_hbm, idx_v, rows_v, acc, sem,
          lsem0, lsem1):
        c = lax.axis_index("c")
        s = lax.axis_index("s")
        is_c0 = c == 0
        lsem = (lsem0, lsem1)

        def load_pair(i, b, issue):
            cb = s * per_t + i * chunk
            rb = s * (per_t // SUB) + i * k_sub
            f = pltpu.async_copy if issue else (
                lambda a, d, sm: pltpu.make_async_copy(a, d, sm))
            return (f(src_hbm.at[pl.ds(rb, k_sub), :], idx_v.at[b], lsem[b]),
                    f(msg_hbm.at[pl.ds(cb, chunk), :], rows_v.at[b], lsem[b]))

        for sweep in range(2):
            pltpu.sync_copy(zeros_hbm, acc.at[pl.ds(s * stripe, stripe), :])
            plsc.subcore_barrier()
            load_pair(0, 0, True)

            def outer_body(i0, carry):
                for b in range(2):
                    i = 2 * i0 + b
                    for d in load_pair(i, b, False):
                        d.wait()

                    @pl.when(i + 1 < n_chunks)
                    def _():
                        load_pair(i + 1, 1 - b, True)

                    def power_loop(fn):
                        def row_body(r, rcarry):
                            rows_v[b, r] = fn(rows_v[b, r])
                            return rcarry
                        lax.fori_loop(0, chunk, row_body, 0, unroll=8)

                    if sweep == 0:
                        @pl.when(jnp.logical_not(is_c0))
                        def _():
                            power_loop(lambda v: v * v)
                    else:
                        @pl.when(is_c0)
                        def _():
                            power_loop(lambda v: v * v * v)

                        @pl.when(jnp.logical_not(is_c0))
                        def _():
                            def quad(v):
                                v2 = v * v
                                return v2 * v2
                            power_loop(quad)

                    descs = [
                        pltpu.async_copy(rows_v.at[b, pl.ds(j * SUB, SUB), :],
                                         acc.at[idx_v.at[b, j]], sem, add=True)
                        for j in range(k_sub)
                    ]
                    for d in descs:
                        d.wait()
                return carry

            lax.fori_loop(0, n_chunks // 2, outer_body, 0)
            plsc.subcore_barrier()
            pltpu.sync_copy(acc.at[pl.ds(s * stripe, stripe), :],
                            out_hbm.at[2 * sweep + c,
                                       pl.ds(s * stripe, stripe), :])

    return k(msg, src2d, zeros)


def kernel(x_s, x_t, edge_index, edge_attr, u, batch_s,
           W1a, b1a, W1b, b1b, W2a, b2a, W2b, b2b):
    n = x_s.shape[0]
    e = edge_attr.shape[0]
    e_pad = -(-e // (32 * CHUNK)) * (32 * CHUNK)
    epk = e_pad // 8
    n_acc = -(-(n + 256) // SUB) * SUB
    pad = e_pad - e

    src = edge_index[0]
    tgt = edge_index[1]
    if pad:
        trash = n + (jnp.arange(pad, dtype=jnp.int32) % (n_acc - n))
        src = jnp.concatenate([src, trash])
        tgt = jnp.concatenate([tgt, jnp.zeros((pad,), jnp.int32)])
    src2d = src.reshape(e_pad // SUB, SUB)
    tgt2d = tgt.reshape(e_pad // SUB, SUB)

    pad_c = lambda w: jnp.pad(w, ((0, 0), (0, LANES - w.shape[1])))
    wy = _kron8(pad_c(W1a[:F_XT]))
    wz = _kron8(pad_c(W1a[F_XT:]))
    wm = _kron8(jnp.pad(W1b, ((0, 1), (0, 1))))
    b1a_pk = jnp.tile(jnp.pad(b1a, (0, 1)), 8).reshape(1, 128)
    b1b_pk = jnp.tile(jnp.pad(b1b, (0, 1), constant_values=1.0), 8).reshape(1, 128)

    full = lambda shape: pl.BlockSpec(shape, lambda i: tuple(0 for _ in shape))

    n8 = n // 8
    xpk = x_t.reshape(n8, 8 * F_XT)
    ypk = pl.pallas_call(
        _prep_y_body,
        grid=(1,),
        in_specs=[pl.BlockSpec((n8, 8 * F_XT), lambda i: (0, 0)),
                  full((8 * F_XT, 128))],
        out_specs=pl.BlockSpec((n8, 128), lambda i: (0, 0)),
        out_shape=jax.ShapeDtypeStruct((n8, 128), jnp.float32),
    )(xpk, wy)
    y = ypk.reshape(n, LANES)

    yg = _gather_rows(y, tgt2d, e_pad)

    attr_pk = edge_attr.reshape(e // 8, 8 * F_E)
    ypk_e = yg.reshape(epk, 128)
    e_blk = 2048
    e_grid = epk // e_blk
    msg_pk = pl.pallas_call(
        _msg_body,
        grid=(e_grid,),
        in_specs=[pl.BlockSpec((e_blk, 8 * F_E), lambda i: (i, 0)),
                  pl.BlockSpec((e_blk, 128), lambda i: (i, 0)),
                  full((8 * F_E, 128)), full((1, 128)),
                  full((128, 128)), full((1, 128))],
        out_specs=pl.BlockSpec((e_blk, 128), lambda i: (i, 0)),
        out_shape=jax.ShapeDtypeStruct((epk, 128), jnp.float32),
    )(attr_pk, ypk_e, wz, b1a_pk, wm, b1b_pk)
    msg = msg_pk.reshape(e_pad, LANES)

    zeros = jnp.zeros((n_acc // 16, LANES), jnp.float32)
    p = _scatter_moments(msg, src2d, zeros, e_pad, n_acc)

    f_xs = x_s.shape[1]
    npk = n_acc // 8
    npk_r = n // 8
    p_pk = p.reshape(4, npk, 128)
    xs_pk = jnp.pad(x_s.reshape(npk_r, 8 * f_xs), ((0, npk - npk_r), (0, 0)))
    bs_pk = jnp.pad(batch_s.astype(jnp.float32).reshape(npk_r, 8),
                    ((0, npk - npk_r), (0, 0)))
    zrow = jnp.zeros((1, 10), jnp.float32)
    pad_r = lambda w: jnp.concatenate([w, zrow], 0)
    selcnt = _kron8(jnp.concatenate([jnp.zeros((15, 16), jnp.float32),
                                     jnp.ones((1, 16), jnp.float32)], 0))
    kcnt = _kron8(jnp.concatenate([jnp.zeros((15, 10), jnp.float32),
                                   W2a[10:11]], 0))
    kmean = _kron8(pad_r(W2a[11:26]))
    kstd = _kron8(pad_r(W2a[26:41]))
    kskew = _kron8(pad_r(W2a[41:56]))
    kkurt = _kron8(pad_r(W2a[56:71]))
    kxs = _kron8(W2a[:f_xs])
    krep = _kron8(jnp.ones((1, 16), jnp.float32))
    iota128 = jnp.tile(jnp.arange(16, dtype=jnp.float32), 8).reshape(1, 128)
    kub = _kron8(u @ W2a[71:81])
    b2a_t = jnp.tile(b2a, 8).reshape(1, -1)
    b2b_t = jnp.tile(b2b, 8).reshape(1, -1)
    kw2b = _kron8(W2b)

    f_blk = 1568
    f_grid = npk // f_blk
    mom_spec = [pl.BlockSpec((1, f_blk, 128), lambda i, _j=j: (_j, i, 0))
                for j in (0, 1, 2, 3)]
    out_pk = pl.pallas_call(
        _final_body,
        grid=(f_grid,),
        in_specs=[pl.BlockSpec((f_blk, 8 * f_xs), lambda i: (i, 0)),
                  mom_spec[0], mom_spec[1], mom_spec[2], mom_spec[3],
                  pl.BlockSpec((f_blk, 8), lambda i: (i, 0)),
                  full(selcnt.shape), full(kcnt.shape), full(kmean.shape),
                  full(kstd.shape), full(kskew.shape), full(kkurt.shape),
                  full(kxs.shape), full(krep.shape), full(iota128.shape),
                  full(kub.shape), full(b2a_t.shape), full(kw2b.shape),
                  full(b2b_t.shape)],
        out_specs=pl.BlockSpec((f_blk, 8 * W2b.shape[1]), lambda i: (i, 0)),
        out_shape=jax.ShapeDtypeStruct((npk, 8 * W2b.shape[1]), jnp.float32),
    )(xs_pk, p_pk, p_pk, p_pk, p_pk, bs_pk, selcnt, kcnt, kmean, kstd,
      kskew, kkurt, kxs, krep, iota128, kub, b2a_t, kw2b, b2b_t)
    return out_pk[:npk_r].reshape(n, W2b.shape[1])


# speedup vs baseline: 16.2230x; 1.0253x over previous
import functools

import jax
import jax.numpy as jnp
from jax import lax
from jax.experimental import pallas as pl
from jax.experimental.pallas import tpu as pltpu
from jax.experimental.pallas import tpu_sc as plsc

F_XT = 5
F_E = 10
D1 = 15
LANES = 16
CHUNK = 1024
SUB = 128
N_BLK = 2000


def _leaky(x):
    return jnp.where(x >= 0, x, 0.1 * x)


def _kron8(w):
    return jnp.kron(jnp.eye(8, dtype=jnp.float32), w)


def _prep_y_body(xpk_ref, wy_ref, ypk_ref):
    ypk_ref[...] = jnp.dot(xpk_ref[...], wy_ref[...],
                           preferred_element_type=jnp.float32)


def _msg_body(attr_ref, ypk_ref, wz_ref, b1a_ref, wm_ref, b1b_ref, out_ref):
    z = jnp.dot(attr_ref[...], wz_ref[...], preferred_element_type=jnp.float32)
    h = _leaky(ypk_ref[...] + z + b1a_ref[...])
    out_ref[...] = (jnp.dot(h, wm_ref[...], preferred_element_type=jnp.float32)
                    + b1b_ref[...])


def _final_body(xs_ref, p1_ref, p2_ref, p3_ref, p4_ref, bs_ref,
                selcnt_ref, kcnt_ref, kmean_ref, kstd_ref, kskew_ref,
                kkurt_ref, kxs_ref, krep_ref, iota_ref, kub_ref,
                b2a_ref, kw2b_ref, b2b_ref, out_ref):
    dot = functools.partial(jnp.dot, preferred_element_type=jnp.float32)
    p1 = p1_ref[0]
    p2 = p2_ref[0]
    p3 = p3_ref[0]
    p4 = p4_ref[0]
    cnt = dot(p1, selcnt_ref[...])
    denom = jnp.maximum(cnt, 1.0)
    mean = p1 / denom
    var = jnp.maximum(p2 / denom - mean * mean, 0.0)
    std = jnp.sqrt(var + 1e-6)
    m2 = mean * mean
    c3 = p3 - 3.0 * mean * p2 + 2.0 * m2 * mean * cnt
    c4 = p4 - 4.0 * mean * p3 + 6.0 * m2 * p2 - 3.0 * m2 * m2 * cnt
    std2 = std * std
    skew = (c3 / denom) / (std2 * std)
    kurt = (c4 / denom) / (std2 * std2)
    bs_rep = dot(bs_ref[...], krep_ref[...])
    onehot = (bs_rep == iota_ref[...]).astype(jnp.float32)
    pre = (dot(xs_ref[...], kxs_ref[...]) + dot(p1, kcnt_ref[...])
           + dot(mean, kmean_ref[...]) + dot(std, kstd_ref[...])
           + dot(skew, kskew_ref[...]) + dot(kurt, kkurt_ref[...])
           + dot(onehot, kub_ref[...]) + b2a_ref[...])
    h1 = _leaky(pre)
    out_ref[...] = dot(h1, kw2b_ref[...]) + b2b_ref[...]


def _sc_mesh():
    return plsc.VectorSubcoreMesh(core_axis_name="c", subcore_axis_name="s",
                                  num_cores=2, num_subcores=16)


def _gather_rows(y, tgt2d, e_pad):
    per_w = e_pad // 32
    n_chunks = per_w // CHUNK
    k_sub = CHUNK // SUB

    @functools.partial(
        pl.kernel,
        out_type=jax.ShapeDtypeStruct((e_pad, LANES), jnp.float32),
        mesh=_sc_mesh(),
        compiler_params=pltpu.CompilerParams(use_tc_tiling_on_sc=False),
        scratch_types=[
            pltpu.VMEM((k_sub, SUB), jnp.int32),
            pltpu.VMEM((CHUNK, LANES), jnp.float32),
            pltpu.SemaphoreType.DMA,
        ],
    )
    def k(y_hbm, tgt_hbm, out_hbm, idx_v, rows_v, sem):
        c = lax.axis_index("c")
        s = lax.axis_index("s")
        wid = s * 2 + c
        base = wid * per_w
        rbase = wid * (per_w // SUB)

        def chunk_body(i, carry):
            cb = base + i * CHUNK
            rb = rbase + i * k_sub
            pltpu.sync_copy(tgt_hbm.at[pl.ds(rb, k_sub), :], idx_v)
            descs = [
                pltpu.async_copy(y_hbm.at[idx_v.at[j]],
                                 rows_v.at[pl.ds(j * SUB, SUB), :], sem)
                for j in range(k_sub)
            ]
            for d in descs:
                d.wait()
            pltpu.sync_copy(rows_v, out_hbm.at[pl.ds(cb, CHUNK), :])
            return carry

        lax.fori_loop(0, n_chunks, chunk_body, 0)

    return k(y, tgt2d)


def _scatter_moments(msg, src2d, zeros, e_pad, n_acc):
    chunk = CHUNK // 2
    per_t = e_pad // 16
    n_chunks = per_t // chunk
    k_sub = chunk // SUB
    stripe = n_acc // 16

    @functools.partial(
        pl.kernel,
        out_type=jax.ShapeDtypeStruct((4, n_acc, LANES), jnp.float32),
        mesh=_sc_mesh(),
        compiler_params=pltpu.CompilerParams(use_tc_tiling_on_sc=False),
        scratch_types=[
            pltpu.VMEM((2, k_sub, SUB), jnp.int32),
            pltpu.VMEM((2, chunk, LANES), jnp.float32),
            pltpu.VMEM_SHARED((n_acc, LANES), jnp.float32),
            pltpu.SemaphoreType.DMA,
            pltpu.SemaphoreType.DMA,
            pltpu.SemaphoreType.DMA,
        ],
    )
    def k(msg_hbm, src_hbm, zeros_hbm, out_hbm, idx_v, rows_v, acc, sem,
          lsem0, lsem1):
        c = lax.axis_index("c")
        s = lax.axis_index("s")
        is_c0 = c == 0
        lsem = (lsem0, lsem1)

        def load_pair(i, b, issue):
            cb = s * per_t + i * chunk
            rb = s * (per_t // SUB) + i * k_sub
            f = pltpu.async_copy if issue else (
                lambda a, d, sm: pltpu.make_async_copy(a, d, sm))
            return (f(src_hbm.at[pl.ds(rb, k_sub), :], idx_v.at[b], lsem[b]),
                    f(msg_hbm.at[pl.ds(cb, chunk), :], rows_v.at[b], lsem[b]))

        for sweep in range(2):
            pltpu.sync_copy(zeros_hbm, acc.at[pl.ds(s * stripe, stripe), :])
            plsc.subcore_barrier()
            load_pair(0, 0, True)

            def outer_body(i0, carry):
                for b in range(2):
                    i = 2 * i0 + b
                    for d in load_pair(i, b, False):
                        d.wait()

                    @pl.when(i + 1 < n_chunks)
                    def _():
                        load_pair(i + 1, 1 - b, True)

                    def power_loop(fn):
                        def row_body(r, rcarry):
                            rows_v[b, r] = fn(rows_v[b, r])
                            return rcarry
                        lax.fori_loop(0, chunk, row_body, 0, unroll=8)

                    if sweep == 0:
                        @pl.when(jnp.logical_not(is_c0))
                        def _():
                            power_loop(lambda v: v * v)
                    else:
                        @pl.when(is_c0)
                        def _():
                            power_loop(lambda v: v * v * v)

                        @pl.when(jnp.logical_not(is_c0))
                        def _():
                            def quad(v):
                                v2 = v * v
                                return v2 * v2
                            power_loop(quad)

                    descs = [
                        pltpu.async_copy(rows_v.at[b, pl.ds(j * SUB, SUB), :],
                                         acc.at[idx_v.at[b, j]], sem, add=True)
                        for j in range(k_sub)
                    ]
                    for d in descs:
                        d.wait()
                return carry

            lax.fori_loop(0, n_chunks // 2, outer_body, 0)
            plsc.subcore_barrier()
            pltpu.sync_copy(acc.at[pl.ds(s * stripe, stripe), :],
                            out_hbm.at[2 * sweep + c,
                                       pl.ds(s * stripe, stripe), :])

    return k(msg, src2d, zeros)


def kernel(x_s, x_t, edge_index, edge_attr, u, batch_s,
           W1a, b1a, W1b, b1b, W2a, b2a, W2b, b2b):
    n = x_s.shape[0]
    e = edge_attr.shape[0]
    e_pad = -(-e // (32 * CHUNK)) * (32 * CHUNK)
    epk = e_pad // 8
    n_acc = -(-(n + 256) // SUB) * SUB
    pad = e_pad - e

    ei2d = edge_index.reshape(2, e // SUB, SUB)
    if pad:
        trash = (n + (jnp.arange(pad, dtype=jnp.int32) % (n_acc - n))
                 ).reshape(1, pad // SUB, SUB)
        pad_blk = jnp.concatenate(
            [trash, jnp.zeros((1, pad // SUB, SUB), jnp.int32)], axis=0)
        ei2d = jnp.concatenate([ei2d, pad_blk], axis=1)
    src2d = ei2d[0]
    tgt2d = ei2d[1]

    pad_c = lambda w: jnp.pad(w, ((0, 0), (0, LANES - w.shape[1])))
    wy = _kron8(pad_c(W1a[:F_XT]))
    wz = _kron8(pad_c(W1a[F_XT:]))
    wm = _kron8(jnp.pad(W1b, ((0, 1), (0, 1))))
    b1a_pk = jnp.tile(jnp.pad(b1a, (0, 1)), 8).reshape(1, 128)
    b1b_pk = jnp.tile(jnp.pad(b1b, (0, 1), constant_values=1.0), 8).reshape(1, 128)

    full = lambda shape: pl.BlockSpec(shape, lambda i: tuple(0 for _ in shape))

    n8 = n // 8
    xpk = x_t.reshape(n8, 8 * F_XT)
    ypk = pl.pallas_call(
        _prep_y_body,
        grid=(1,),
        in_specs=[pl.BlockSpec((n8, 8 * F_XT), lambda i: (0, 0)),
                  full((8 * F_XT, 128))],
        out_specs=pl.BlockSpec((n8, 128), lambda i: (0, 0)),
        out_shape=jax.ShapeDtypeStruct((n8, 128), jnp.float32),
    )(xpk, wy)
    y = ypk.reshape(n, LANES)

    yg = _gather_rows(y, tgt2d, e_pad)

    attr_pk = edge_attr.reshape(e // 8, 8 * F_E)
    ypk_e = yg.reshape(epk, 128)
    e_blk = 2048
    e_grid = epk // e_blk
    msg_pk = pl.pallas_call(
        _msg_body,
        grid=(e_grid,),
        in_specs=[pl.BlockSpec((e_blk, 8 * F_E), lambda i: (i, 0)),
                  pl.BlockSpec((e_blk, 128), lambda i: (i, 0)),
                  full((8 * F_E, 128)), full((1, 128)),
                  full((128, 128)), full((1, 128))],
        out_specs=pl.BlockSpec((e_blk, 128), lambda i: (i, 0)),
        out_shape=jax.ShapeDtypeStruct((epk, 128), jnp.float32),
    )(attr_pk, ypk_e, wz, b1a_pk, wm, b1b_pk)
    msg = msg_pk.reshape(e_pad, LANES)

    zeros = jnp.zeros((n_acc // 16, LANES), jnp.float32)
    p = _scatter_moments(msg, src2d, zeros, e_pad, n_acc)

    f_xs = x_s.shape[1]
    npk = n_acc // 8
    npk_r = n // 8
    p_pk = p.reshape(4, npk, 128)
    xs_pk = jnp.pad(x_s.reshape(npk_r, 8 * f_xs), ((0, npk - npk_r), (0, 0)))
    bs_pk = jnp.pad(batch_s.astype(jnp.float32).reshape(npk_r, 8),
                    ((0, npk - npk_r), (0, 0)))
    zrow = jnp.zeros((1, 10), jnp.float32)
    pad_r = lambda w: jnp.concatenate([w, zrow], 0)
    selcnt = _kron8(jnp.concatenate([jnp.zeros((15, 16), jnp.float32),
                                     jnp.ones((1, 16), jnp.float32)], 0))
    kcnt = _kron8(jnp.concatenate([jnp.zeros((15, 10), jnp.float32),
                                   W2a[10:11]], 0))
    kmean = _kron8(pad_r(W2a[11:26]))
    kstd = _kron8(pad_r(W2a[26:41]))
    kskew = _kron8(pad_r(W2a[41:56]))
    kkurt = _kron8(pad_r(W2a[56:71]))
    kxs = _kron8(W2a[:f_xs])
    krep = _kron8(jnp.ones((1, 16), jnp.float32))
    iota128 = jnp.tile(jnp.arange(16, dtype=jnp.float32), 8).reshape(1, 128)
    kub = _kron8(u @ W2a[71:81])
    b2a_t = jnp.tile(b2a, 8).reshape(1, -1)
    b2b_t = jnp.tile(b2b, 8).reshape(1, -1)
    kw2b = _kron8(W2b)

    f_blk = 1568
    f_grid = npk // f_blk
    mom_spec = [pl.BlockSpec((1, f_blk, 128), lambda i, _j=j: (_j, i, 0))
                for j in (0, 1, 2, 3)]
    out_pk = pl.pallas_call(
        _final_body,
        grid=(f_grid,),
        in_specs=[pl.BlockSpec((f_blk, 8 * f_xs), lambda i: (i, 0)),
                  mom_spec[0], mom_spec[1], mom_spec[2], mom_spec[3],
                  pl.BlockSpec((f_blk, 8), lambda i: (i, 0)),
                  full(selcnt.shape), full(kcnt.shape), full(kmean.shape),
                  full(kstd.shape), full(kskew.shape), full(kkurt.shape),
                  full(kxs.shape), full(krep.shape), full(iota128.shape),
                  full(kub.shape), full(b2a_t.shape), full(kw2b.shape),
                  full(b2b_t.shape)],
        out_specs=pl.BlockSpec((f_blk, 8 * W2b.shape[1]), lambda i: (i, 0)),
        out_shape=jax.ShapeDtypeStruct((npk, 8 * W2b.shape[1]), jnp.float32),
    )(xs_pk, p_pk, p_pk, p_pk, p_pk, bs_pk, selcnt, kcnt, kmean, kstd,
      kskew, kkurt, kxs, krep, iota128, kub, b2a_t, kw2b, b2b_t)
    return out_pk[:npk_r].reshape(n, W2b.shape[1])
